# Initial kernel scaffold; baseline (speedup 1.0000x reference)
import jax
import jax.numpy as jnp
from jax.experimental import pallas as pl


def kernel(x, edge_index, los, params):
    raise NotImplementedError("write your pallas kernel here")


# speedup vs baseline: 7.7815x; 7.7815x over previous
import functools

import jax
import jax.numpy as jnp
import numpy as np
from jax import lax
from jax.experimental import pallas as pl
from jax.experimental.pallas import tpu as pltpu, tpu_sc as plsc

N = 50000
NPAD = 51200
E = 800000
EPAD = 819200
D = 64
HD = 32
BN = 1000
GRID = N // BN
HB = GRID // 2
NOUT = 50048

@functools.cache
def _get_mesh():
    return plsc.VectorSubcoreMesh(core_axis_name="c", subcore_axis_name="s")


def _zero_fill(zbuf):
    z = jnp.zeros((16,), jnp.float32)

    def zrow(j, c):
        zbuf[j, pl.ds(0, 16)] = z
        zbuf[j, pl.ds(16, 16)] = z
        return c

    lax.fori_loop(0, zbuf.shape[0], zrow, 0)


def _emb_kernel(e_lo, e_hi, idx2d, out_lo, out_hi, idxbuf, rows):
    cid = lax.axis_index("c")
    sid = lax.axis_index("s")
    pltpu.sync_copy(idx2d.at[sid], idxbuf)

    def run(e_ref, out_ref):
        def gat(j, c):
            pltpu.sync_copy(e_ref.at[idxbuf.at[j]], rows)
            pltpu.sync_copy(rows, out_ref.at[pl.ds(sid * 3200 + j * 128, 128), :])
            return c

        lax.fori_loop(0, 25, gat, 0)

    pl.when(cid == 0)(lambda: run(e_lo, out_lo))
    pl.when(cid == 1)(lambda: run(e_hi, out_hi))


def _emb_gather(e_lo, e_hi, idx2d):
    f = pl.kernel(
        _emb_kernel,
        out_type=(
            jax.ShapeDtypeStruct((NPAD, HD), jnp.float32),
            jax.ShapeDtypeStruct((NPAD, HD), jnp.float32),
        ),
        mesh=_get_mesh(),
        compiler_params=pltpu.CompilerParams(use_tc_tiling_on_sc=False),
        scratch_types=[
            pltpu.VMEM((25, 128), jnp.int32),
            pltpu.VMEM((128, HD), jnp.float32),
        ],
    )
    return f(e_lo, e_hi, idx2d)


def _aggr_kernel(t_lo, t_hi, src2d, dst2d, out_lo, out_hi,
                 sidx, didx, rows, zbuf, acc):
    cid = lax.axis_index("c")
    sid = lax.axis_index("s")

    _zero_fill(zbuf)

    def zcp(w, c):
        pltpu.sync_copy(zbuf, acc.at[pl.ds(sid * 3200 + w * 128, 128), :])
        return c

    lax.fori_loop(0, 25, zcp, 0)
    plsc.subcore_barrier()

    def scatter_phase(t_ref):
        base = sid * 8
        for blk in range(8):
            pltpu.sync_copy(src2d.at[base + blk], sidx)
            pltpu.sync_copy(dst2d.at[base + blk], didx)

            def step(j, c):
                pltpu.sync_copy(t_ref.at[sidx.at[j]], rows)
                pltpu.sync_copy(rows, acc.at[didx.at[j]], add=True)
                return c

            lax.fori_loop(0, 50, step, 0)

    pl.when(cid == 0)(lambda: scatter_phase(t_lo))
    pl.when(cid == 1)(lambda: scatter_phase(t_hi))
    plsc.subcore_barrier()

    def outcopy(out_ref):
        pltpu.sync_copy(acc.at[pl.ds(sid * 3128, 3128), :],
                        out_ref.at[pl.ds(sid * 3128, 3128), :])

    pl.when(cid == 0)(lambda: outcopy(out_lo))
    pl.when(cid == 1)(lambda: outcopy(out_hi))


def _aggr(t_lo, t_hi, src2d, dst2d):
    f = pl.kernel(
        _aggr_kernel,
        out_type=(
            jax.ShapeDtypeStruct((NOUT, HD), jnp.float32),
            jax.ShapeDtypeStruct((NOUT, HD), jnp.float32),
        ),
        mesh=_get_mesh(),
        compiler_params=pltpu.CompilerParams(use_tc_tiling_on_sc=False),
        scratch_types=[
            pltpu.VMEM((50, 128), jnp.int32),
            pltpu.VMEM((50, 128), jnp.int32),
            pltpu.VMEM((128, HD), jnp.float32),
            pltpu.VMEM((128, HD), jnp.float32),
            pltpu.VMEM_SHARED((NPAD, HD), jnp.float32),
        ],
    )
    return f(t_lo, t_hi, src2d, dst2d)


def _mlp_body(flags, *refs):
    has_cross, pool, make_g, out_y = flags
    it = iter(refs)
    hlo, hhi, alo, ahi, scale = (next(it) for _ in range(5))
    if has_cross:
        hslo, hshi, los2d, ltc, wec, bec = (next(it) for _ in range(6))
    w1, b1, lng, lnb, w2, b2 = (next(it) for _ in range(6))
    if make_g:
        ltn, wen, ben = (next(it) for _ in range(3))
    if out_y:
        ylo, yhi = next(it), next(it)
    if make_g:
        glo, ghi = next(it), next(it)
    if pool:
        pout = next(it)

    h = jnp.concatenate([hlo[...], hhi[...]], axis=1)
    a = jnp.concatenate([alo[...], ahi[...]], axis=1)
    u = h * scale[0, 0] + a
    if has_cross:
        i = pl.program_id(0)
        hs = jnp.concatenate([hslo[...], hshi[...]], axis=1)
        vl = jnp.dot(ltc[...], wec[...], preferred_element_type=jnp.float32) + bec[...]
        lb = los2d[...][0, 0]
        oh = (lb[:, None] == lax.broadcasted_iota(jnp.int32, (BN // 10, 38), 1)
              ).astype(jnp.float32)
        vrow = jnp.dot(oh, vl, preferred_element_type=jnp.float32)
        rep = (lax.broadcasted_iota(jnp.int32, (BN, BN // 10), 0) // 10 ==
               lax.broadcasted_iota(jnp.int32, (BN, BN // 10), 1)).astype(jnp.float32)
        vrep = jnp.dot(rep, vrow, preferred_element_type=jnp.float32)
        mask = jnp.where(i >= HB, 1.0, 0.0)
        u = u + mask * jnp.maximum(hs + vrep, 0.0)
    z = jnp.dot(u, w1[...], preferred_element_type=jnp.float32) + b1[...]
    mu = jnp.mean(z, axis=1, keepdims=True)
    var = jnp.mean((z - mu) ** 2, axis=1, keepdims=True)
    z = (z - mu) * lax.rsqrt(var + 1e-5) * lng[...] + lnb[...]
    z = jnp.maximum(z, 0.0)
    y = jnp.dot(z, w2[...], preferred_element_type=jnp.float32) + b2[...]
    if out_y:
        ylo[...] = y[:, :HD]
        yhi[...] = y[:, HD:]
    if make_g:
        v0 = jnp.dot(ltn[...][0:1, :], wen[...],
                     preferred_element_type=jnp.float32) + ben[...]
        gg = jnp.maximum(y + v0, 0.0)
        glo[...] = gg[:, :HD]
        ghi[...] = gg[:, HD:]
    if pool:
        pm = (lax.broadcasted_iota(jnp.int32, (BN // 10, BN), 0) ==
              lax.broadcasted_iota(jnp.int32, (BN // 10, BN), 1) // 10
              ).astype(jnp.float32)
        pout[...] = jnp.dot(pm, y, preferred_element_type=jnp.float32)[None]


def _mlp_call(hlo, hhi, alo, ahi, scale, mats, cross=None, gparams=None,
              pool=False, out_y=True):
    has_cross = cross is not None
    make_g = gparams is not None
    flags = (has_cross, pool, make_g, out_y)

    node_spec = pl.BlockSpec((BN, HD), lambda i: (i, 0))
    full = lambda s: pl.BlockSpec(s, lambda i: (0, 0))
    shift_spec = pl.BlockSpec((BN, HD), lambda i: (jnp.maximum(i - HB, 0), 0))

    args = [hlo, hhi, alo, ahi, scale]
    in_specs = [node_spec, node_spec, node_spec, node_spec, full((1, 1))]
    if has_cross:
        hslo, hshi, los2d, ltc, wec, bec = cross
        args += [hslo, hshi, los2d, ltc, wec, bec]
        in_specs += [shift_spec, shift_spec,
                     pl.BlockSpec((1, 1, BN // 10),
                                  lambda i: (jnp.maximum(i - HB, 0), 0, 0)),
                     full((38, 8)), full((8, D)), full((1, D))]
    w1, b1, lng, lnb, w2, b2 = mats
    args += [w1, b1, lng, lnb, w2, b2]
    in_specs += [full((D, D)), full((1, D)), full((1, D)), full((1, D)),
                 full((D, D)), full((1, D))]
    if make_g:
        ltn, wen, ben = gparams
        args += [ltn, wen, ben]
        in_specs += [full((38, 8)), full((8, D)), full((1, D))]

    out_shapes = []
    out_specs = []
    if out_y:
        out_shapes += [jax.ShapeDtypeStruct((N, HD), jnp.float32)] * 2
        out_specs += [node_spec, node_spec]
    if make_g:
        out_shapes += [jax.ShapeDtypeStruct((N, HD), jnp.float32)] * 2
        out_specs += [node_spec, node_spec]
    if pool:
        out_shapes += [jax.ShapeDtypeStruct((GRID, BN // 10, D), jnp.float32)]
        out_specs += [pl.BlockSpec((1, BN // 10, D), lambda i: (i, 0, 0))]

    f = pl.pallas_call(
        functools.partial(_mlp_body, flags),
        grid=(GRID,),
        in_specs=in_specs,
        out_specs=out_specs,
        out_shape=out_shapes,
    )
    return f(*args)


def kernel(x, edge_index, los, params):
    p = params
    offs = jnp.asarray(
        np.concatenate([[0], np.cumsum([1000] * 20)[:-1]]).astype(np.int32))
    flat_idx = (x.astype(jnp.int32) + offs[None, :]).reshape(-1)
    pad_i = jnp.arange(NPAD - N, dtype=jnp.int32) % 20000
    idx2d = jnp.concatenate([flat_idx, pad_i]).reshape(16, 25, 128)

    et = p['emb_table']
    e_lo, e_hi = et[:, :HD], et[:, HD:]

    src = edge_index[0].astype(jnp.int32)
    dst = edge_index[1].astype(jnp.int32)
    pe = jnp.arange(EPAD - E, dtype=jnp.int32)
    src2d = jnp.concatenate([src, pe % N]).reshape(128, 50, 128)
    dst2d = jnp.concatenate([dst, N + (pe % 128)]).reshape(128, 50, 128)
    los2d = los.astype(jnp.int32).reshape(HB, 1, BN // 10)

    def mats(pre):
        return (p[pre + '_W1'], p[pre + '_b1'].reshape(1, D),
                p[pre + '_ln_g'].reshape(1, D), p[pre + '_ln_b'].reshape(1, D),
                p[pre + '_W2'], p[pre + '_b2'].reshape(1, D))

    def sc(pre):
        return (1.0 + p[pre + '_eps']).reshape(1, 1)

    lt = p['los_table']
    edge0 = (lt, p['g2_0_We'], p['g2_0_be'].reshape(1, D))
    edge1 = (lt, p['g2_1_We'], p['g2_1_be'].reshape(1, D))

    h0lo, h0hi = _emb_gather(e_lo, e_hi, idx2d)
    a0lo, a0hi = _aggr(h0lo, h0hi, src2d, dst2d)
    h1lo, h1hi = _mlp_call(h0lo, h0hi, a0lo, a0hi, sc('g1_0'), mats('g1_0'))
    a1lo, a1hi = _aggr(h1lo, h1hi, src2d, dst2d)
    h2lo, h2hi, g0lo, g0hi, ad_dis = _mlp_call(
        h1lo, h1hi, a1lo, a1hi, sc('g1_1'), mats('g1_1'),
        gparams=edge0, pool=True)
    a2lo, a2hi = _aggr(g0lo, g0hi, src2d, dst2d)
    h3lo, h3hi, g1lo, g1hi = _mlp_call(
        h2lo, h2hi, a2lo, a2hi, sc('g2_0'), mats('g2_0'),
        cross=(h2lo, h2hi, los2d) + edge0, gparams=edge1)
    a3lo, a3hi = _aggr(g1lo, g1hi, src2d, dst2d)
    (x_sum,) = _mlp_call(
        h3lo, h3hi, a3lo, a3hi, sc('g2_1'), mats('g2_1'),
        cross=(h3lo, h3hi, los2d) + edge1, pool=True, out_y=False)
    return (ad_dis.reshape(N // 10, D), x_sum.reshape(N // 10, D))
